# baseline (speedup 1.0000x reference)
import functools

import jax
import jax.numpy as jnp
from jax import lax
from jax.experimental import pallas as pl
from jax.experimental.pallas import tpu as pltpu
from jax.experimental.pallas import tpu_sc as plsc

BATCH = 16384
DIM = 64
NUM_CORES = 2
NUM_SUBCORES = 16
NUM_WORKERS = NUM_CORES * NUM_SUBCORES
PER_WORKER = BATCH // NUM_WORKERS
CHUNK = 128
NUM_CHUNKS = PER_WORKER // CHUNK
LANES = 16
VREGS_PER_ROW = DIM // LANES


def _vec_sqrt(x):
    i = plsc.bitcast(x, jnp.int32)
    i = jnp.int32(0x5F3759DF) - lax.shift_right_logical(i, 1)
    y = plsc.bitcast(i, jnp.float32)
    half = x * jnp.float32(0.5)
    for _ in range(3):
        y = y * (jnp.float32(1.5) - half * y * y)
    return x * y


def _transe_body(heads_hbm, rels_hbm, tails_hbm, ent_hbm, relw_hbm, out_hbm,
                 hidx_v, ridx_v, tidx_v, hrow_v, rrow_v, trow_v, out_v, sem):
    wid = lax.axis_index("s") * NUM_CORES + lax.axis_index("c")
    base_w = wid * PER_WORKER

    for c in range(NUM_CHUNKS):
        base = base_w + c * CHUNK
        pltpu.sync_copy(heads_hbm.at[pl.ds(base, CHUNK)], hidx_v)
        pltpu.sync_copy(rels_hbm.at[pl.ds(base, CHUNK)], ridx_v)
        pltpu.sync_copy(tails_hbm.at[pl.ds(base, CHUNK)], tidx_v)
        ch = pltpu.async_copy(ent_hbm.at[hidx_v], hrow_v, sem)
        cr = pltpu.async_copy(relw_hbm.at[ridx_v], rrow_v, sem)
        ct = pltpu.async_copy(ent_hbm.at[tidx_v], trow_v, sem)
        ch.wait()
        cr.wait()
        ct.wait()

        def group(g, _, c=c):
            base = g * LANES
            res = jnp.zeros((LANES,), jnp.float32)
            for j in range(LANES):
                i = base + j
                acc = jnp.zeros((LANES,), jnp.float32)
                for d in range(VREGS_PER_ROW):
                    hv = hrow_v[i, pl.ds(d * LANES, LANES)]
                    rv = rrow_v[i, pl.ds(d * LANES, LANES)]
                    tv = trow_v[i, pl.ds(d * LANES, LANES)]
                    diff = (hv - tv) + rv
                    acc = acc + diff * diff
                lane_j = jnp.arange(LANES, dtype=jnp.int32) == j
                res = jnp.where(lane_j, jnp.sum(acc), res)
            out_v[pl.ds(c * CHUNK + base, LANES)] = _vec_sqrt(res)
            return 0

        lax.fori_loop(0, CHUNK // LANES, group, 0)

    pltpu.sync_copy(out_v, out_hbm.at[pl.ds(base_w, PER_WORKER)])


@functools.partial(jax.jit, static_argnames=())
def _transe(heads, relations, tails, entity_weight, relation_weight):
    mesh = plsc.VectorSubcoreMesh(core_axis_name="c", subcore_axis_name="s")
    f = functools.partial(
        pl.kernel,
        out_type=jax.ShapeDtypeStruct((BATCH,), jnp.float32),
        mesh=mesh,
        scratch_types=[
            pltpu.VMEM((CHUNK,), jnp.int32),
            pltpu.VMEM((CHUNK,), jnp.int32),
            pltpu.VMEM((CHUNK,), jnp.int32),
            pltpu.VMEM((CHUNK, DIM), jnp.float32),
            pltpu.VMEM((CHUNK, DIM), jnp.float32),
            pltpu.VMEM((CHUNK, DIM), jnp.float32),
            pltpu.VMEM((PER_WORKER,), jnp.float32),
            pltpu.SemaphoreType.DMA,
        ],
        compiler_params=pltpu.CompilerParams(
            needs_layout_passes=False, use_tc_tiling_on_sc=False
        ),
    )(_transe_body)
    return f(heads, relations, tails, entity_weight, relation_weight)


def kernel(heads, relations, tails, entity_weight, relation_weight):
    return _transe(heads, relations, tails, entity_weight, relation_weight)


# speedup vs baseline: 1.7306x; 1.7306x over previous
import functools

import jax
import jax.numpy as jnp
from jax import lax
from jax.experimental import pallas as pl
from jax.experimental.pallas import tpu as pltpu
from jax.experimental.pallas import tpu_sc as plsc

BATCH = 16384
DIM = 64
SUB = 8
NUM_CORES = 2
NUM_SUBCORES = 16
NUM_WORKERS = NUM_CORES * NUM_SUBCORES
PER_WORKER = BATCH // NUM_WORKERS
LANES = 16
NUM_GROUPS = PER_WORKER // LANES


def _vec_sqrt(x):
    i = plsc.bitcast(x, jnp.int32)
    i = jnp.int32(0x5F3759DF) - lax.shift_right_logical(i, 1)
    y = plsc.bitcast(i, jnp.float32)
    half = x * jnp.float32(0.5)
    for _ in range(3):
        y = y * (jnp.float32(1.5) - half * y * y)
    return x * y


def _transe_body(heads_hbm, rels_hbm, tails_hbm, ent_hbm, relw_hbm, out_hbm,
                 hraw_v, rraw_v, traw_v, hbuf_v, rbuf_v, tbuf_v, out_v, sem):
    wid = lax.axis_index("s") * NUM_CORES + lax.axis_index("c")
    base_w = wid * PER_WORKER
    pltpu.sync_copy(heads_hbm.at[pl.ds(base_w, PER_WORKER)], hraw_v)
    pltpu.sync_copy(rels_hbm.at[pl.ds(base_w, PER_WORKER)], rraw_v)
    pltpu.sync_copy(tails_hbm.at[pl.ds(base_w, PER_WORKER)], traw_v)

    lane_iota = lax.iota(jnp.int32, LANES)

    def group(g, _):
        goff = pl.multiple_of(g * LANES, LANES)
        sl = pl.ds(goff, LANES)
        hraw = hraw_v[sl]
        rraw = rraw_v[sl]
        traw = traw_v[sl]
        hblk = lax.shift_right_logical(hraw, 3)
        rblk = lax.shift_right_logical(rraw, 3)
        tblk = lax.shift_right_logical(traw, 3)
        for j in range(LANES):
            lane_j = lane_iota == j
            th = jnp.sum(jnp.where(lane_j, hblk, 0))
            tr = jnp.sum(jnp.where(lane_j, rblk, 0))
            tt = jnp.sum(jnp.where(lane_j, tblk, 0))
            pltpu.async_copy(ent_hbm.at[th], hbuf_v.at[pl.ds(j * SUB, SUB), :], sem)
            pltpu.async_copy(relw_hbm.at[tr], rbuf_v.at[pl.ds(j * SUB, SUB), :], sem)
            pltpu.async_copy(ent_hbm.at[tt], tbuf_v.at[pl.ds(j * SUB, SUB), :], sem)
        for j in range(LANES):
            pltpu.make_async_copy(ent_hbm.at[0], hbuf_v.at[pl.ds(j * SUB, SUB), :], sem).wait()
            pltpu.make_async_copy(relw_hbm.at[0], rbuf_v.at[pl.ds(j * SUB, SUB), :], sem).wait()
            pltpu.make_async_copy(ent_hbm.at[0], tbuf_v.at[pl.ds(j * SUB, SUB), :], sem).wait()

        seven = jnp.full((LANES,), SUB - 1, jnp.int32)
        hrow = lane_iota * SUB + lax.bitwise_and(hraw, seven)
        rrow = lane_iota * SUB + lax.bitwise_and(rraw, seven)
        trow = lane_iota * SUB + lax.bitwise_and(traw, seven)

        def dim_step(d, acc, hrow=hrow, rrow=rrow, trow=trow):
            hv = plsc.load_gather(hbuf_v, [hrow, hrow * 0 + d])
            rv = plsc.load_gather(rbuf_v, [rrow, rrow * 0 + d])
            tv = plsc.load_gather(tbuf_v, [trow, trow * 0 + d])
            diff = (hv - tv) + rv
            return acc + diff * diff

        acc = lax.fori_loop(0, DIM, dim_step, jnp.zeros((LANES,), jnp.float32))
        out_v[sl] = _vec_sqrt(acc)
        return 0

    lax.fori_loop(0, NUM_GROUPS, group, 0)
    pltpu.sync_copy(out_v, out_hbm.at[pl.ds(base_w, PER_WORKER)])


@jax.jit
def _transe(heads, relations, tails, entity_weight, relation_weight):
    ent3 = entity_weight.reshape(entity_weight.shape[0] // SUB, SUB, DIM)
    rel3 = relation_weight.reshape(relation_weight.shape[0] // SUB, SUB, DIM)
    mesh = plsc.VectorSubcoreMesh(core_axis_name="c", subcore_axis_name="s")
    f = functools.partial(
        pl.kernel,
        out_type=jax.ShapeDtypeStruct((BATCH,), jnp.float32),
        mesh=mesh,
        scratch_types=[
            pltpu.VMEM((PER_WORKER,), jnp.int32),
            pltpu.VMEM((PER_WORKER,), jnp.int32),
            pltpu.VMEM((PER_WORKER,), jnp.int32),
            pltpu.VMEM((LANES * SUB, DIM), jnp.float32),
            pltpu.VMEM((LANES * SUB, DIM), jnp.float32),
            pltpu.VMEM((LANES * SUB, DIM), jnp.float32),
            pltpu.VMEM((PER_WORKER,), jnp.float32),
            pltpu.SemaphoreType.DMA,
        ],
        compiler_params=pltpu.CompilerParams(
            needs_layout_passes=False, use_tc_tiling_on_sc=True
        ),
    )(_transe_body)
    return f(heads, relations, tails, ent3, rel3)


def kernel(heads, relations, tails, entity_weight, relation_weight):
    return _transe(heads, relations, tails, entity_weight, relation_weight)


# speedup vs baseline: 2.0061x; 1.1592x over previous
import functools

import jax
import jax.numpy as jnp
from jax import lax
from jax.experimental import pallas as pl
from jax.experimental.pallas import tpu as pltpu
from jax.experimental.pallas import tpu_sc as plsc

BATCH = 16384
DIM = 64
SUB = 8
NUM_CORES = 2
NUM_SUBCORES = 16
NUM_WORKERS = NUM_CORES * NUM_SUBCORES
PER_WORKER = BATCH // NUM_WORKERS
LANES = 16
CHUNK = 128
NUM_CHUNKS = PER_WORKER // CHUNK
GROUPS = CHUNK // LANES
REL_ROWS = 500


def _vec_sqrt(x):
    i = plsc.bitcast(x, jnp.int32)
    i = jnp.int32(0x5F3759DF) - lax.shift_right_logical(i, 1)
    y = plsc.bitcast(i, jnp.float32)
    half = x * jnp.float32(0.5)
    for _ in range(3):
        y = y * (jnp.float32(1.5) - half * y * y)
    return x * y


def _transe_body(heads_hbm, rels_hbm, tails_hbm, ent_hbm, relw_hbm, out_hbm,
                 hraw_v, rraw_v, traw_v, rel_v,
                 hbufs, tbufs, out_v, sems):
    wid = lax.axis_index("s") * NUM_CORES + lax.axis_index("c")
    base_w = wid * PER_WORKER
    pltpu.sync_copy(relw_hbm, rel_v)

    lane_iota = lax.iota(jnp.int32, LANES)
    seven = jnp.full((LANES,), SUB - 1, jnp.int32)
    one = jnp.full((LANES,), 1, jnp.int32)

    def issue(g, s):
        sl = pl.ds(g * LANES, LANES)
        hblk = lax.shift_right_logical(hraw_v[sl], 3)
        tblk = lax.shift_right_logical(traw_v[sl], 3)
        for j in range(LANES):
            pltpu.async_copy(ent_hbm.at[hblk[j]],
                             hbufs[s].at[pl.ds(j * SUB, SUB), :], sems[s])
            pltpu.async_copy(ent_hbm.at[tblk[j]],
                             tbufs[s].at[pl.ds(j * SUB, SUB), :], sems[s])

    def drain(s):
        pltpu.make_async_copy(ent_hbm.at[0], hbufs[s], sems[s]).wait()
        pltpu.make_async_copy(ent_hbm.at[0], tbufs[s], sems[s]).wait()

    def compute(g, s, c):
        sl = pl.ds(g * LANES, LANES)
        hraw = hraw_v[sl]
        rraw = rraw_v[sl]
        traw = traw_v[sl]
        hrow = lane_iota * SUB + lax.bitwise_and(hraw, seven)
        trow = lane_iota * SUB + lax.bitwise_and(traw, seven)
        rrow = lax.shift_right_logical(rraw, 1)
        rcol = lax.bitwise_and(rraw, one) * DIM

        def dim_step(d, acc):
            hv = plsc.load_gather(hbufs[s], [hrow, hrow * 0 + d])
            tv = plsc.load_gather(tbufs[s], [trow, trow * 0 + d])
            rv = plsc.load_gather(rel_v, [rrow, rcol + d])
            diff = (hv - tv) + rv
            return acc + diff * diff

        acc = lax.fori_loop(0, DIM, dim_step, jnp.zeros((LANES,), jnp.float32))
        out_v[pl.ds(c * CHUNK + g * LANES, LANES)] = _vec_sqrt(acc)

    for c in range(NUM_CHUNKS):
        base = base_w + c * CHUNK
        pltpu.sync_copy(heads_hbm.at[pl.ds(base, CHUNK)], hraw_v)
        pltpu.sync_copy(rels_hbm.at[pl.ds(base, CHUNK)], rraw_v)
        pltpu.sync_copy(tails_hbm.at[pl.ds(base, CHUNK)], traw_v)
        issue(0, 0)

        def pair(k, _, c=c):
            g0 = k * 2
            drain(0)
            issue(g0 + 1, 1)
            compute(g0, 0, c)
            drain(1)

            @pl.when(k < GROUPS // 2 - 1)
            def _():
                issue(g0 + 2, 0)

            compute(g0 + 1, 1, c)
            return 0

        lax.fori_loop(0, GROUPS // 2, pair, 0)

    pltpu.sync_copy(out_v, out_hbm.at[pl.ds(base_w, PER_WORKER)])


@jax.jit
def _transe(heads, relations, tails, entity_weight, relation_weight):
    ent3 = entity_weight.reshape(entity_weight.shape[0] // SUB, SUB, DIM)
    rel2 = relation_weight.reshape(relation_weight.shape[0] // 2, 2 * DIM)
    mesh = plsc.VectorSubcoreMesh(core_axis_name="c", subcore_axis_name="s")
    f = functools.partial(
        pl.kernel,
        out_type=jax.ShapeDtypeStruct((BATCH,), jnp.float32),
        mesh=mesh,
        scratch_types=[
            pltpu.VMEM((CHUNK,), jnp.int32),
            pltpu.VMEM((CHUNK,), jnp.int32),
            pltpu.VMEM((CHUNK,), jnp.int32),
            pltpu.VMEM((REL_ROWS, 2 * DIM), jnp.float32),
            [pltpu.VMEM((LANES * SUB, DIM), jnp.float32) for _ in range(2)],
            [pltpu.VMEM((LANES * SUB, DIM), jnp.float32) for _ in range(2)],
            pltpu.VMEM((PER_WORKER,), jnp.float32),
            [pltpu.SemaphoreType.DMA for _ in range(2)],
        ],
        compiler_params=pltpu.CompilerParams(
            needs_layout_passes=False, use_tc_tiling_on_sc=True
        ),
    )(_transe_body)
    return f(heads, relations, tails, ent3, rel2)


def kernel(heads, relations, tails, entity_weight, relation_weight):
    return _transe(heads, relations, tails, entity_weight, relation_weight)


# speedup vs baseline: 2.1014x; 1.0475x over previous
import functools

import jax
import jax.numpy as jnp
from jax import lax
from jax.experimental import pallas as pl
from jax.experimental.pallas import tpu as pltpu
from jax.experimental.pallas import tpu_sc as plsc

BATCH = 16384
DIM = 64
SUB = 8
NUM_CORES = 2
NUM_SUBCORES = 16
NUM_WORKERS = NUM_CORES * NUM_SUBCORES
PER_WORKER = BATCH // NUM_WORKERS
LANES = 16
CHUNK = 128
NUM_CHUNKS = PER_WORKER // CHUNK
GROUPS = CHUNK // LANES
REL_ROWS = 500


def _vec_sqrt(x):
    i = plsc.bitcast(x, jnp.int32)
    i = jnp.int32(0x5F3759DF) - lax.shift_right_logical(i, 1)
    y = plsc.bitcast(i, jnp.float32)
    half = x * jnp.float32(0.5)
    for _ in range(3):
        y = y * (jnp.float32(1.5) - half * y * y)
    return x * y


def _transe_body(heads_hbm, rels_hbm, tails_hbm, ent_hbm, relw_hbm, out_hbm,
                 hraw_v, rraw_v, traw_v, rel_v,
                 hbufs, tbufs, out_v, sems):
    wid = lax.axis_index("s") * NUM_CORES + lax.axis_index("c")
    base_w = wid * PER_WORKER
    pltpu.sync_copy(relw_hbm, rel_v)

    lane_iota = lax.iota(jnp.int32, LANES)
    one = jnp.full((LANES,), 1, jnp.int32)

    def issue(g, s):
        sl = pl.ds(g * LANES, LANES)
        hraw = hraw_v[sl]
        traw = traw_v[sl]
        for j in range(LANES):
            hr = hraw[j]
            tr = traw[j]
            pltpu.async_copy(
                ent_hbm.at[lax.shift_right_logical(hr, 3),
                           lax.bitwise_and(hr, SUB - 1), :],
                hbufs[s].at[j // SUB, j % SUB], sems[s])
            pltpu.async_copy(
                ent_hbm.at[lax.shift_right_logical(tr, 3),
                           lax.bitwise_and(tr, SUB - 1), :],
                tbufs[s].at[j // SUB, j % SUB], sems[s])

    def drain(s):
        pltpu.make_async_copy(ent_hbm.at[pl.ds(0, LANES // SUB)],
                              hbufs[s], sems[s]).wait()
        pltpu.make_async_copy(ent_hbm.at[pl.ds(0, LANES // SUB)],
                              tbufs[s], sems[s]).wait()

    def compute(g, s, c):
        sl = pl.ds(g * LANES, LANES)
        rraw = rraw_v[sl]
        rrow = lax.shift_right_logical(rraw, 1)
        rcol = lax.bitwise_and(rraw, one) * DIM

        row_hi = lax.shift_right_logical(lane_iota, 3)
        row_lo = lax.bitwise_and(lane_iota, jnp.full((LANES,), SUB - 1, jnp.int32))

        def dim_step(d, acc, rrow=rrow, rcol=rcol, s=s):
            hv = plsc.load_gather(hbufs[s], [row_hi, row_lo, row_hi * 0 + d])
            tv = plsc.load_gather(tbufs[s], [row_hi, row_lo, row_hi * 0 + d])
            rv = plsc.load_gather(rel_v, [rrow, rcol + d])
            diff = (hv - tv) + rv
            return acc + diff * diff

        acc = lax.fori_loop(0, DIM, dim_step, jnp.zeros((LANES,), jnp.float32))
        out_v[pl.ds(c * CHUNK + g * LANES, LANES)] = _vec_sqrt(acc)

    for c in range(NUM_CHUNKS):
        base = base_w + c * CHUNK
        pltpu.sync_copy(heads_hbm.at[pl.ds(base, CHUNK)], hraw_v)
        pltpu.sync_copy(rels_hbm.at[pl.ds(base, CHUNK)], rraw_v)
        pltpu.sync_copy(tails_hbm.at[pl.ds(base, CHUNK)], traw_v)
        issue(0, 0)

        def pair(k, _, c=c):
            g0 = k * 2
            drain(0)
            issue(g0 + 1, 1)
            compute(g0, 0, c)
            drain(1)

            @pl.when(k < GROUPS // 2 - 1)
            def _():
                issue(g0 + 2, 0)

            compute(g0 + 1, 1, c)
            return 0

        lax.fori_loop(0, GROUPS // 2, pair, 0)

    pltpu.sync_copy(out_v, out_hbm.at[pl.ds(base_w, PER_WORKER)])


@jax.jit
def _transe(heads, relations, tails, entity_weight, relation_weight):
    ent3 = entity_weight.reshape(entity_weight.shape[0] // SUB, SUB, DIM)
    rel2 = relation_weight.reshape(relation_weight.shape[0] // 2, 2 * DIM)
    mesh = plsc.VectorSubcoreMesh(core_axis_name="c", subcore_axis_name="s")
    f = functools.partial(
        pl.kernel,
        out_type=jax.ShapeDtypeStruct((BATCH,), jnp.float32),
        mesh=mesh,
        scratch_types=[
            pltpu.VMEM((CHUNK,), jnp.int32),
            pltpu.VMEM((CHUNK,), jnp.int32),
            pltpu.VMEM((CHUNK,), jnp.int32),
            pltpu.VMEM((REL_ROWS, 2 * DIM), jnp.float32),
            [pltpu.VMEM((LANES // SUB, SUB, DIM), jnp.float32) for _ in range(2)],
            [pltpu.VMEM((LANES // SUB, SUB, DIM), jnp.float32) for _ in range(2)],
            pltpu.VMEM((PER_WORKER,), jnp.float32),
            [pltpu.SemaphoreType.DMA for _ in range(2)],
        ],
        compiler_params=pltpu.CompilerParams(
            needs_layout_passes=False, use_tc_tiling_on_sc=True
        ),
    )(_transe_body)
    return f(heads, relations, tails, ent3, rel2)


def kernel(heads, relations, tails, entity_weight, relation_weight):
    return _transe(heads, relations, tails, entity_weight, relation_weight)


# speedup vs baseline: 2.1528x; 1.0245x over previous
import functools

import jax
import jax.numpy as jnp
from jax import lax
from jax.experimental import pallas as pl
from jax.experimental.pallas import tpu as pltpu
from jax.experimental.pallas import tpu_sc as plsc

BATCH = 16384
DIM = 64
SUB = 8
NUM_CORES = 2
NUM_SUBCORES = 16
NUM_WORKERS = NUM_CORES * NUM_SUBCORES
PER_WORKER = BATCH // NUM_WORKERS
LANES = 16
NUM_GROUPS = PER_WORKER // LANES
NSETS = 4
REL_ROWS = 500


def _vec_sqrt(x):
    i = plsc.bitcast(x, jnp.int32)
    i = jnp.int32(0x5F3759DF) - lax.shift_right_logical(i, 1)
    y = plsc.bitcast(i, jnp.float32)
    half = x * jnp.float32(0.5)
    for _ in range(3):
        y = y * (jnp.float32(1.5) - half * y * y)
    return x * y


def _transe_body(heads_hbm, rels_hbm, tails_hbm, ent_hbm, relw_hbm, out_hbm,
                 hraw_v, rraw_v, traw_v, rel_v,
                 hbufs, tbufs, out_v, sems):
    wid = lax.axis_index("s") * NUM_CORES + lax.axis_index("c")
    base_w = wid * PER_WORKER
    pltpu.sync_copy(relw_hbm, rel_v)
    pltpu.sync_copy(heads_hbm.at[pl.ds(base_w, PER_WORKER)], hraw_v)
    pltpu.sync_copy(rels_hbm.at[pl.ds(base_w, PER_WORKER)], rraw_v)
    pltpu.sync_copy(tails_hbm.at[pl.ds(base_w, PER_WORKER)], traw_v)

    lane_iota = lax.iota(jnp.int32, LANES)
    one = jnp.full((LANES,), 1, jnp.int32)
    row_hi = lax.shift_right_logical(lane_iota, 3)
    row_lo = lax.bitwise_and(lane_iota, jnp.full((LANES,), SUB - 1, jnp.int32))

    def issue(g, s):
        sl = pl.ds(g * LANES, LANES)
        hraw = hraw_v[sl]
        traw = traw_v[sl]
        for j in range(LANES):
            hr = hraw[j]
            tr = traw[j]
            pltpu.async_copy(
                ent_hbm.at[lax.shift_right_logical(hr, 3),
                           lax.bitwise_and(hr, SUB - 1), :],
                hbufs[s].at[j // SUB, j % SUB], sems[s])
            pltpu.async_copy(
                ent_hbm.at[lax.shift_right_logical(tr, 3),
                           lax.bitwise_and(tr, SUB - 1), :],
                tbufs[s].at[j // SUB, j % SUB], sems[s])

    def drain(s):
        pltpu.make_async_copy(ent_hbm.at[pl.ds(0, LANES // SUB)],
                              hbufs[s], sems[s]).wait()
        pltpu.make_async_copy(ent_hbm.at[pl.ds(0, LANES // SUB)],
                              tbufs[s], sems[s]).wait()

    def compute(g, s):
        sl = pl.ds(g * LANES, LANES)
        rraw = rraw_v[sl]
        rrow = lax.shift_right_logical(rraw, 1)
        rcol = lax.bitwise_and(rraw, one) * DIM

        def dim_step(d, acc, rrow=rrow, rcol=rcol, s=s):
            hv = plsc.load_gather(hbufs[s], [row_hi, row_lo, row_hi * 0 + d])
            tv = plsc.load_gather(tbufs[s], [row_hi, row_lo, row_hi * 0 + d])
            rv = plsc.load_gather(rel_v, [rrow, rcol + d])
            diff = (hv - tv) + rv
            return acc + diff * diff

        acc = lax.fori_loop(0, DIM, dim_step, jnp.zeros((LANES,), jnp.float32))
        out_v[sl] = _vec_sqrt(acc)

    for s in range(NSETS - 1):
        issue(s, s)

    def step(k, _):
        for u in range(NSETS):
            g = k * NSETS + u
            drain(u)

            @pl.when(g + NSETS - 1 < NUM_GROUPS)
            def _(g=g, u=u):
                issue(g + NSETS - 1, (u + NSETS - 1) % NSETS)

            compute(g, u)
        return 0

    lax.fori_loop(0, NUM_GROUPS // NSETS, step, 0)
    pltpu.sync_copy(out_v, out_hbm.at[pl.ds(base_w, PER_WORKER)])


@jax.jit
def _transe(heads, relations, tails, entity_weight, relation_weight):
    ent3 = entity_weight.reshape(entity_weight.shape[0] // SUB, SUB, DIM)
    rel2 = relation_weight.reshape(relation_weight.shape[0] // 2, 2 * DIM)
    mesh = plsc.VectorSubcoreMesh(core_axis_name="c", subcore_axis_name="s")
    f = functools.partial(
        pl.kernel,
        out_type=jax.ShapeDtypeStruct((BATCH,), jnp.float32),
        mesh=mesh,
        scratch_types=[
            pltpu.VMEM((PER_WORKER,), jnp.int32),
            pltpu.VMEM((PER_WORKER,), jnp.int32),
            pltpu.VMEM((PER_WORKER,), jnp.int32),
            pltpu.VMEM((REL_ROWS, 2 * DIM), jnp.float32),
            [pltpu.VMEM((LANES // SUB, SUB, DIM), jnp.float32)
             for _ in range(NSETS)],
            [pltpu.VMEM((LANES // SUB, SUB, DIM), jnp.float32)
             for _ in range(NSETS)],
            pltpu.VMEM((PER_WORKER,), jnp.float32),
            [pltpu.SemaphoreType.DMA for _ in range(NSETS)],
        ],
        compiler_params=pltpu.CompilerParams(
            needs_layout_passes=False, use_tc_tiling_on_sc=True
        ),
    )(_transe_body)
    return f(heads, relations, tails, ent3, rel2)


def kernel(heads, relations, tails, entity_weight, relation_weight):
    return _transe(heads, relations, tails, entity_weight, relation_weight)
